# Initial kernel scaffold; baseline (speedup 1.0000x reference)
import functools

import jax
import jax.numpy as jnp
from jax import lax
from jax.experimental import pallas as pl
from jax.experimental.pallas import tpu as pltpu
from jax.experimental.pallas import tpu_sc as plsc

NC = 2
NS = 16
NW = NC * NS
CH = 128
D = 128


def _sc_agg_body(n_chunks, per_w, n_nodes, agg_rows,
                 h_hbm, src_hbm, dst_hbm, out_hbm,
                 src_v, dst_v, msg_v, zero_v, agg_sh, sem):
    c = lax.axis_index("c")
    s = lax.axis_index("s")
    wid = c * NS + s

    zrows = agg_rows // NS

    def _zero_row(i, _):
        for j in range(D // 16):
            zero_v[i, pl.ds(j * 16, 16)] = jnp.zeros((16,), jnp.float32)
        return 0

    lax.fori_loop(0, zrows, _zero_row, 0)
    pltpu.sync_copy(zero_v, agg_sh.at[pl.ds(s * zrows, zrows)])
    plsc.subcore_barrier()

    base0 = wid * per_w

    def _chunk(j, _):
        base = base0 + j * CH
        pltpu.sync_copy(src_hbm.at[pl.ds(base, CH)], src_v)
        pltpu.sync_copy(dst_hbm.at[pl.ds(base, CH)], dst_v)
        pltpu.async_copy(h_hbm.at[src_v], msg_v, sem).wait()
        pltpu.sync_copy(msg_v, agg_sh.at[dst_v], add=True)
        return 0

    lax.fori_loop(0, n_chunks, _chunk, 0)
    plsc.subcore_barrier()

    wrows = n_nodes // NS
    pltpu.sync_copy(agg_sh.at[pl.ds(s * wrows, wrows)],
                    out_hbm.at[c, pl.ds(s * wrows, wrows)])


@functools.partial(jax.jit, static_argnums=(3,))
def _sc_agg(h, src_pad, dst_pad, n_nodes):
    e_pad = src_pad.shape[0]
    per_w = e_pad // NW
    n_chunks = per_w // CH
    agg_rows = ((n_nodes + 1 + NS - 1) // NS) * NS
    mesh = plsc.VectorSubcoreMesh(core_axis_name="c", subcore_axis_name="s",
                                  num_cores=NC, num_subcores=NS)
    body = functools.partial(_sc_agg_body, n_chunks, per_w, n_nodes, agg_rows)
    kern = pl.kernel(
        body,
        out_type=jax.ShapeDtypeStruct((NC, n_nodes, D), jnp.float32),
        mesh=mesh,
        scratch_types=[
            pltpu.VMEM((CH,), jnp.int32),
            pltpu.VMEM((CH,), jnp.int32),
            pltpu.VMEM((CH, D), jnp.float32),
            pltpu.VMEM((agg_rows // NS, D), jnp.float32),
            pltpu.VMEM_SHARED((agg_rows, D), jnp.float32),
            pltpu.SemaphoreType.DMA,
        ],
    )
    return kern(h, src_pad, dst_pad)


def _tc_body(h_ref, a0_ref, a1_ref, w_ref, o_ref):
    agg = a0_ref[...] + a1_ref[...]
    t = jnp.dot(agg, w_ref[...], preferred_element_type=jnp.float32)
    o_ref[...] = h_ref[...] + jnp.maximum(t, 0.0)


def _tc_update(h, a0, a1, w):
    n = h.shape[0]
    blk = 1000
    grid = (n // blk,)
    return pl.pallas_call(
        _tc_body,
        grid=grid,
        in_specs=[
            pl.BlockSpec((blk, D), lambda i: (i, 0)),
            pl.BlockSpec((blk, D), lambda i: (i, 0)),
            pl.BlockSpec((blk, D), lambda i: (i, 0)),
            pl.BlockSpec((D, D), lambda i: (0, 0)),
        ],
        out_specs=pl.BlockSpec((blk, D), lambda i: (i, 0)),
        out_shape=jax.ShapeDtypeStruct((n, D), jnp.float32),
    )(h, a0, a1, w)


def kernel(x, edge_index, W):
    n = x.shape[0]
    e = edge_index.shape[1]
    src = edge_index[0].astype(jnp.int32)
    dst = edge_index[1].astype(jnp.int32)

    per_w = ((e + NW - 1) // NW + CH - 1) // CH * CH
    e_pad = per_w * NW
    src_pad = jnp.concatenate(
        [src, jnp.zeros((e_pad - e,), jnp.int32)]) if e_pad > e else src
    dst_pad = jnp.concatenate(
        [dst, jnp.full((e_pad - e,), n, jnp.int32)]) if e_pad > e else dst

    h = x
    for l in range(W.shape[0]):
        agg2 = _sc_agg(h, src_pad, dst_pad, n)
        h = _tc_update(h, agg2[0], agg2[1], W[l])
    return h


# speedup vs baseline: 3.8937x; 3.8937x over previous
import functools

import jax
import jax.numpy as jnp
from jax import lax
from jax.experimental import pallas as pl
from jax.experimental.pallas import tpu as pltpu
from jax.experimental.pallas import tpu_sc as plsc

NC = 2
NS = 16
NW = NC * NS
CH = 128
D = 128


def _sc_agg_body(n_chunks, per_w, agg_rows,
                 h_hbm, src_hbm, dst_hbm, out_hbm,
                 src_v, dst_v, msg_v, agg_sh, sem):
    c = lax.axis_index("c")
    s = lax.axis_index("s")
    wid = c * NS + s

    zrows = agg_rows // NS

    def _zero_row(i, _):
        for j in range(D // 16):
            msg_v[i, pl.ds(j * 16, 16)] = jnp.zeros((16,), jnp.float32)
        return 0

    lax.fori_loop(0, CH, _zero_row, 0)
    for k in range(zrows // CH):
        pltpu.sync_copy(msg_v, agg_sh.at[pl.ds(s * zrows + k * CH, CH)])
    rem = zrows % CH
    if rem:
        pltpu.sync_copy(msg_v.at[pl.ds(0, rem)],
                        agg_sh.at[pl.ds(s * zrows + (zrows // CH) * CH, rem)])
    plsc.subcore_barrier()

    base0 = wid * per_w

    def _chunk(j, _):
        base = base0 + j * CH
        pltpu.sync_copy(src_hbm.at[pl.ds(base, CH)], src_v)
        pltpu.sync_copy(dst_hbm.at[pl.ds(base, CH)], dst_v)
        pltpu.async_copy(h_hbm.at[src_v], msg_v, sem).wait()
        pltpu.sync_copy(msg_v, agg_sh.at[dst_v], add=True)
        return 0

    lax.fori_loop(0, n_chunks, _chunk, 0)
    plsc.subcore_barrier()

    pltpu.sync_copy(agg_sh.at[pl.ds(s * zrows, zrows)],
                    out_hbm.at[c, pl.ds(s * zrows, zrows)])


@functools.partial(jax.jit, static_argnums=(3,))
def _sc_agg(h, src_pad, dst_pad, n_nodes):
    e_pad = src_pad.shape[0]
    per_w = e_pad // NW
    n_chunks = per_w // CH
    agg_rows = ((n_nodes + 1 + NS * 8 - 1) // (NS * 8)) * (NS * 8)
    mesh = plsc.VectorSubcoreMesh(core_axis_name="c", subcore_axis_name="s",
                                  num_cores=NC, num_subcores=NS)
    body = functools.partial(_sc_agg_body, n_chunks, per_w, agg_rows)
    kern = pl.kernel(
        body,
        out_type=jax.ShapeDtypeStruct((NC, agg_rows, D), jnp.float32),
        mesh=mesh,
        scratch_types=[
            pltpu.VMEM((CH,), jnp.int32),
            pltpu.VMEM((CH,), jnp.int32),
            pltpu.VMEM((CH, D), jnp.float32),
            pltpu.VMEM_SHARED((agg_rows, D), jnp.float32),
            pltpu.SemaphoreType.DMA,
        ],
    )
    return kern(h, src_pad, dst_pad)


def _tc_body(h_ref, a0_ref, a1_ref, w_ref, o_ref):
    agg = a0_ref[0] + a1_ref[0]
    t = jnp.dot(agg, w_ref[...], preferred_element_type=jnp.float32)
    o_ref[...] = h_ref[...] + jnp.maximum(t, 0.0)


def _tc_update(h, agg2, w):
    n = h.shape[0]
    blk = 1000
    grid = (n // blk,)
    return pl.pallas_call(
        _tc_body,
        grid=grid,
        in_specs=[
            pl.BlockSpec((blk, D), lambda i: (i, 0)),
            pl.BlockSpec((1, blk, D), lambda i: (0, i, 0)),
            pl.BlockSpec((1, blk, D), lambda i: (1, i, 0)),
            pl.BlockSpec((D, D), lambda i: (0, 0)),
        ],
        out_specs=pl.BlockSpec((blk, D), lambda i: (i, 0)),
        out_shape=jax.ShapeDtypeStruct((n, D), jnp.float32),
    )(h, agg2, agg2, w)


def kernel(x, edge_index, W):
    n = x.shape[0]
    e = edge_index.shape[1]
    src = edge_index[0].astype(jnp.int32)
    dst = edge_index[1].astype(jnp.int32)

    per_w = ((e + NW - 1) // NW + CH - 1) // CH * CH
    e_pad = per_w * NW
    src_pad = jnp.concatenate(
        [src, jnp.zeros((e_pad - e,), jnp.int32)]) if e_pad > e else src
    dst_pad = jnp.concatenate(
        [dst, jnp.full((e_pad - e,), n, jnp.int32)]) if e_pad > e else dst

    h = x
    for l in range(W.shape[0]):
        agg2 = _sc_agg(h, src_pad, dst_pad, n)
        h = _tc_update(h, agg2, W[l])
    return h
